# baseline (speedup 1.0000x reference)
import functools

import jax
import jax.numpy as jnp
from jax import lax
from jax.experimental import pallas as pl
from jax.experimental.pallas import tpu as pltpu
from jax.experimental.pallas import tpu_sc as plsc

VOCAB = 1000000
EMB_DIM = 64
SEQ = 32
BATCH = 4096
HIDDEN = 128

NC = 2
NS = 16
NW = NC * NS
TOTAL = BATCH * SEQ
B_PER_W = TOTAL // NW
CH = 512
NCH = B_PER_W // CH


def _sc_gather_body(table_hbm, idx_hbm, out_hbm, idx_v, rows_v, gsem, wsem):
    wid = lax.axis_index("s") * NC + lax.axis_index("c")
    base = wid * B_PER_W
    pltpu.sync_copy(idx_hbm.at[wid], idx_v)

    pltpu.async_copy(table_hbm.at[idx_v.at[0]], rows_v.at[0], gsem.at[0])

    def chunk(j, _):
        b = lax.rem(j, 2)
        nb = 1 - b
        pltpu.make_async_copy(
            table_hbm.at[pl.ds(0, CH)], rows_v.at[b], gsem.at[b]).wait()

        @pl.when(j + 1 < NCH)
        def _():
            @pl.when(j >= 1)
            def _():
                pltpu.make_async_copy(
                    rows_v.at[nb],
                    out_hbm.at[pl.ds(base + (j - 1) * CH, CH)],
                    wsem.at[nb]).wait()

            pltpu.async_copy(
                table_hbm.at[idx_v.at[j + 1]], rows_v.at[nb], gsem.at[nb])

        pltpu.async_copy(
            rows_v.at[b], out_hbm.at[pl.ds(base + j * CH, CH)], wsem.at[b])
        return 0

    lax.fori_loop(0, NCH, chunk, 0)
    pltpu.make_async_copy(
        rows_v.at[(NCH - 2) % 2],
        out_hbm.at[pl.ds(base + (NCH - 2) * CH, CH)],
        wsem.at[(NCH - 2) % 2]).wait()
    pltpu.make_async_copy(
        rows_v.at[(NCH - 1) % 2],
        out_hbm.at[pl.ds(base + (NCH - 1) * CH, CH)],
        wsem.at[(NCH - 1) % 2]).wait()


@functools.cache
def _make_sc_gather():
    mesh = plsc.VectorSubcoreMesh(
        core_axis_name="c", subcore_axis_name="s", num_cores=NC, num_subcores=NS
    )
    return pl.kernel(
        _sc_gather_body,
        out_type=jax.ShapeDtypeStruct((TOTAL, EMB_DIM), jnp.float32),
        mesh=mesh,
        scratch_types=[
            pltpu.VMEM((NCH, CH), jnp.int32),
            pltpu.VMEM((2, CH, EMB_DIM), jnp.float32),
            pltpu.SemaphoreType.DMA((2,)),
            pltpu.SemaphoreType.DMA((2,)),
        ],
        compiler_params=pltpu.CompilerParams(use_tc_tiling_on_sc=False),
    )


BB = 256


def _mlp_body(x_ref, w1_ref, b1_ref, w2_ref, b2_ref, w3_ref, b3_ref,
              w4_ref, b4_ref, o_ref):
    x = x_ref[...]
    h = jax.nn.sigmoid(
        jnp.dot(x, w1_ref[...], preferred_element_type=jnp.float32) + b1_ref[...])
    h = jax.nn.sigmoid(
        jnp.dot(h, w2_ref[...], preferred_element_type=jnp.float32) + b2_ref[...])
    h = jax.nn.sigmoid(
        jnp.dot(h, w3_ref[...], preferred_element_type=jnp.float32) + b3_ref[...])
    r = jnp.sum(h * w4_ref[...], axis=1, keepdims=True) + b4_ref[...]
    o_ref[...] = jax.nn.sigmoid(r)


def _mlp(x, W1, b1, W2, b2, W3, b3, w4row, b4):
    din = SEQ * EMB_DIM
    return pl.pallas_call(
        _mlp_body,
        grid=(BATCH // BB,),
        in_specs=[
            pl.BlockSpec((BB, din), lambda i: (i, 0)),
            pl.BlockSpec((din, EMB_DIM), lambda i: (0, 0)),
            pl.BlockSpec((1, EMB_DIM), lambda i: (0, 0)),
            pl.BlockSpec((EMB_DIM, HIDDEN), lambda i: (0, 0)),
            pl.BlockSpec((1, HIDDEN), lambda i: (0, 0)),
            pl.BlockSpec((HIDDEN, HIDDEN), lambda i: (0, 0)),
            pl.BlockSpec((1, HIDDEN), lambda i: (0, 0)),
            pl.BlockSpec((1, HIDDEN), lambda i: (0, 0)),
            pl.BlockSpec((1, 1), lambda i: (0, 0)),
        ],
        out_specs=pl.BlockSpec((BB, 1), lambda i: (i, 0)),
        out_shape=jax.ShapeDtypeStruct((BATCH, 1), jnp.float32),
    )(x, W1, b1, W2, b2, W3, b3, w4row, b4)


def kernel(inputs, emb, W1, b1, W2, b2, W3, b3, W4, b4):
    idx = inputs.astype(jnp.int32).reshape(NW, NCH, CH)
    gathered = _make_sc_gather()(emb, idx)
    return gathered
    x = gathered.reshape(BATCH, SEQ * EMB_DIM)
    return _mlp(
        x, W1, b1.reshape(1, EMB_DIM), W2, b2.reshape(1, HIDDEN),
        W3, b3.reshape(1, HIDDEN), W4.reshape(1, HIDDEN), b4.reshape(1, 1),
    )


# speedup vs baseline: 1.0336x; 1.0336x over previous
import functools

import jax
import jax.numpy as jnp
from jax import lax
from jax.experimental import pallas as pl
from jax.experimental.pallas import tpu as pltpu
from jax.experimental.pallas import tpu_sc as plsc

VOCAB = 1000000
EMB_DIM = 64
SEQ = 32
BATCH = 4096
HIDDEN = 128

NC = 2
NS = 16
NW = NC * NS
TOTAL = BATCH * SEQ
B_PER_W = TOTAL // NW
CH = 512
NCH = B_PER_W // CH


def _sc_gather_body(table_hbm, idx_hbm, out_hbm, idx_v, rows_v, gsem, wsem):
    wid = lax.axis_index("s") * NC + lax.axis_index("c")
    base = wid * B_PER_W
    pltpu.sync_copy(idx_hbm.at[wid], idx_v)

    pltpu.async_copy(table_hbm.at[idx_v.at[0]], rows_v.at[0], gsem.at[0])

    def chunk(j, _):
        b = lax.rem(j, 2)
        nb = 1 - b
        pltpu.make_async_copy(
            table_hbm.at[pl.ds(0, CH)], rows_v.at[b], gsem.at[b]).wait()

        @pl.when(j + 1 < NCH)
        def _():
            @pl.when(j >= 1)
            def _():
                pltpu.make_async_copy(
                    rows_v.at[nb],
                    out_hbm.at[pl.ds(base + (j - 1) * CH, CH)],
                    wsem.at[nb]).wait()

            pltpu.async_copy(
                table_hbm.at[idx_v.at[j + 1]], rows_v.at[nb], gsem.at[nb])

        pltpu.async_copy(
            rows_v.at[b], out_hbm.at[pl.ds(base + j * CH, CH)], wsem.at[b])
        return 0

    lax.fori_loop(0, NCH, chunk, 0)
    pltpu.make_async_copy(
        rows_v.at[(NCH - 2) % 2],
        out_hbm.at[pl.ds(base + (NCH - 2) * CH, CH)],
        wsem.at[(NCH - 2) % 2]).wait()
    pltpu.make_async_copy(
        rows_v.at[(NCH - 1) % 2],
        out_hbm.at[pl.ds(base + (NCH - 1) * CH, CH)],
        wsem.at[(NCH - 1) % 2]).wait()


@functools.cache
def _make_sc_gather():
    mesh = plsc.VectorSubcoreMesh(
        core_axis_name="c", subcore_axis_name="s", num_cores=NC, num_subcores=NS
    )
    return pl.kernel(
        _sc_gather_body,
        out_type=jax.ShapeDtypeStruct((TOTAL, EMB_DIM), jnp.float32),
        mesh=mesh,
        scratch_types=[
            pltpu.VMEM((NCH, CH), jnp.int32),
            pltpu.VMEM((2, CH, EMB_DIM), jnp.float32),
            pltpu.SemaphoreType.DMA((2,)),
            pltpu.SemaphoreType.DMA((2,)),
        ],
        compiler_params=pltpu.CompilerParams(use_tc_tiling_on_sc=False),
    )


BB = 256


def _mlp_body(x_ref, w1_ref, b1_ref, w2_ref, b2_ref, w3_ref, b3_ref,
              w4_ref, b4_ref, o_ref):
    x = x_ref[...]
    h = jax.nn.sigmoid(
        jnp.dot(x, w1_ref[...], preferred_element_type=jnp.float32) + b1_ref[...])
    h = jax.nn.sigmoid(
        jnp.dot(h, w2_ref[...], preferred_element_type=jnp.float32) + b2_ref[...])
    h = jax.nn.sigmoid(
        jnp.dot(h, w3_ref[...], preferred_element_type=jnp.float32) + b3_ref[...])
    r = jnp.sum(h * w4_ref[...], axis=1, keepdims=True) + b4_ref[...]
    o_ref[...] = jax.nn.sigmoid(r)


def _mlp(x, W1, b1, W2, b2, W3, b3, w4row, b4):
    din = SEQ * EMB_DIM
    return pl.pallas_call(
        _mlp_body,
        grid=(BATCH // BB,),
        in_specs=[
            pl.BlockSpec((BB, din), lambda i: (i, 0)),
            pl.BlockSpec((din, EMB_DIM), lambda i: (0, 0)),
            pl.BlockSpec((1, EMB_DIM), lambda i: (0, 0)),
            pl.BlockSpec((EMB_DIM, HIDDEN), lambda i: (0, 0)),
            pl.BlockSpec((1, HIDDEN), lambda i: (0, 0)),
            pl.BlockSpec((HIDDEN, HIDDEN), lambda i: (0, 0)),
            pl.BlockSpec((1, HIDDEN), lambda i: (0, 0)),
            pl.BlockSpec((1, HIDDEN), lambda i: (0, 0)),
            pl.BlockSpec((1, 1), lambda i: (0, 0)),
        ],
        out_specs=pl.BlockSpec((BB, 1), lambda i: (i, 0)),
        out_shape=jax.ShapeDtypeStruct((BATCH, 1), jnp.float32),
    )(x, W1, b1, W2, b2, W3, b3, w4row, b4)


def kernel(inputs, emb, W1, b1, W2, b2, W3, b3, W4, b4):
    idx = inputs.astype(jnp.int32).reshape(NW, NCH, CH)
    gathered = _make_sc_gather()(emb, idx)
    x = gathered.reshape(BATCH, SEQ * EMB_DIM)
    return _mlp(
        x, W1, b1.reshape(1, EMB_DIM), W2, b2.reshape(1, HIDDEN),
        W3, b3.reshape(1, HIDDEN), W4.reshape(1, HIDDEN), b4.reshape(1, 1),
    )


# speedup vs baseline: 1.4874x; 1.4391x over previous
import functools

import jax
import jax.numpy as jnp
from jax import lax
from jax.experimental import pallas as pl
from jax.experimental.pallas import tpu as pltpu
from jax.experimental.pallas import tpu_sc as plsc

VOCAB = 1000000
EMB_DIM = 64
SEQ = 32
BATCH = 4096
HIDDEN = 128

NC = 2
NS = 16
NW = NC * NS
TOTAL = BATCH * SEQ
B_PER_W = TOTAL // NW
CH = 256
NCH = B_PER_W // CH


def _sc_gather_body(table_hbm, idx_hbm, out_hbm, idx_t, rows_v, gsem, wsem):
    wid = lax.axis_index("s") * NC + lax.axis_index("c")
    base = wid * B_PER_W
    pltpu.sync_copy(idx_hbm.at[wid], idx_t)

    def chunk(j, _):
        b = lax.rem(j, 2)

        @pl.when(j >= 2)
        def _():
            pltpu.make_async_copy(
                rows_v.at[b], out_hbm.at[pl.ds(base, CH)], wsem.at[b]).wait()

        def group(g, _):
            v = idx_t[j, pl.ds(g * 16, 16)]
            for e in range(16):
                pltpu.async_copy(
                    table_hbm.at[pl.ds(v[e], 1)],
                    rows_v.at[b, pl.ds(g * 16 + e, 1)],
                    gsem.at[b])
            return 0
        lax.fori_loop(0, CH // 16, group, 0)

        def rwait(k, _):
            pltpu.make_async_copy(
                table_hbm.at[pl.ds(0, 1)], rows_v.at[b, pl.ds(0, 1)],
                gsem.at[b]).wait()
            return 0
        lax.fori_loop(0, CH, rwait, 0)

        pltpu.async_copy(
            rows_v.at[b], out_hbm.at[pl.ds(base + j * CH, CH)], wsem.at[b])
        return 0

    lax.fori_loop(0, NCH, chunk, 0)
    pltpu.make_async_copy(
        rows_v.at[(NCH - 2) % 2],
        out_hbm.at[pl.ds(base + (NCH - 2) * CH, CH)],
        wsem.at[(NCH - 2) % 2]).wait()
    pltpu.make_async_copy(
        rows_v.at[(NCH - 1) % 2],
        out_hbm.at[pl.ds(base + (NCH - 1) * CH, CH)],
        wsem.at[(NCH - 1) % 2]).wait()


@functools.cache
def _make_sc_gather():
    mesh = plsc.VectorSubcoreMesh(
        core_axis_name="c", subcore_axis_name="s", num_cores=NC, num_subcores=NS
    )
    return pl.kernel(
        _sc_gather_body,
        out_type=jax.ShapeDtypeStruct((TOTAL, EMB_DIM), jnp.float32),
        mesh=mesh,
        scratch_types=[
            pltpu.VMEM((NCH, CH), jnp.int32),
            pltpu.VMEM((2, CH, EMB_DIM), jnp.float32),
            pltpu.SemaphoreType.DMA((2,)),
            pltpu.SemaphoreType.DMA((2,)),
        ],
    )


BB = 256


def _mlp_body(x_ref, w1_ref, b1_ref, w2_ref, b2_ref, w3_ref, b3_ref,
              w4_ref, b4_ref, o_ref):
    x = x_ref[...]
    h = jax.nn.sigmoid(
        jnp.dot(x, w1_ref[...], preferred_element_type=jnp.float32) + b1_ref[...])
    h = jax.nn.sigmoid(
        jnp.dot(h, w2_ref[...], preferred_element_type=jnp.float32) + b2_ref[...])
    h = jax.nn.sigmoid(
        jnp.dot(h, w3_ref[...], preferred_element_type=jnp.float32) + b3_ref[...])
    r = jnp.sum(h * w4_ref[...], axis=1, keepdims=True) + b4_ref[...]
    o_ref[...] = jax.nn.sigmoid(r)


def _mlp(x, W1, b1, W2, b2, W3, b3, w4row, b4):
    din = SEQ * EMB_DIM
    return pl.pallas_call(
        _mlp_body,
        grid=(BATCH // BB,),
        in_specs=[
            pl.BlockSpec((BB, din), lambda i: (i, 0)),
            pl.BlockSpec((din, EMB_DIM), lambda i: (0, 0)),
            pl.BlockSpec((1, EMB_DIM), lambda i: (0, 0)),
            pl.BlockSpec((EMB_DIM, HIDDEN), lambda i: (0, 0)),
            pl.BlockSpec((1, HIDDEN), lambda i: (0, 0)),
            pl.BlockSpec((HIDDEN, HIDDEN), lambda i: (0, 0)),
            pl.BlockSpec((1, HIDDEN), lambda i: (0, 0)),
            pl.BlockSpec((1, HIDDEN), lambda i: (0, 0)),
            pl.BlockSpec((1, 1), lambda i: (0, 0)),
        ],
        out_specs=pl.BlockSpec((BB, 1), lambda i: (i, 0)),
        out_shape=jax.ShapeDtypeStruct((BATCH, 1), jnp.float32),
    )(x, W1, b1, W2, b2, W3, b3, w4row, b4)


def kernel(inputs, emb, W1, b1, W2, b2, W3, b3, W4, b4):
    idx = inputs.astype(jnp.int32).reshape(NW, NCH, CH)
    gathered = _make_sc_gather()(emb, idx)
    x = gathered.reshape(BATCH, SEQ * EMB_DIM)
    return _mlp(
        x, W1, b1.reshape(1, EMB_DIM), W2, b2.reshape(1, HIDDEN),
        W3, b3.reshape(1, HIDDEN), W4.reshape(1, HIDDEN), b4.reshape(1, 1),
    )
